# baseline (speedup 1.0000x reference)
import functools
import math

import jax
import jax.numpy as jnp
from jax import lax
from jax.experimental import pallas as pl
from jax.experimental.pallas import tpu as pltpu
from jax.experimental.pallas import tpu_sc as plsc

B = 16384
D = 128
L = 16
NC = 2
NS = 16
NW = NC * NS
BPW = B // NW
CHUNK = 64
NCHUNK = BPW // CHUNK
VPR = D // L


def _make_sc_fn():
  mesh = plsc.VectorSubcoreMesh(core_axis_name="c", subcore_axis_name="s")

  @functools.partial(
      pl.kernel,
      mesh=mesh,
      out_type=jax.ShapeDtypeStruct((NW, L), jnp.float32),
      scratch_types=[
          pltpu.VMEM((BPW,), jnp.int32),
          pltpu.VMEM((BPW, D), jnp.float32),
          pltpu.VMEM((CHUNK, D), jnp.float32),
          pltpu.VMEM((CHUNK, D), jnp.float32),
          pltpu.VMEM((CHUNK, D), jnp.float32),
          pltpu.VMEM((CHUNK, D), jnp.float32),
          pltpu.VMEM((CHUNK, D), jnp.float32),
          pltpu.VMEM((CHUNK, D), jnp.float32),
          pltpu.VMEM((CHUNK, D), jnp.float32),
          pltpu.VMEM((L,), jnp.float32),
          pltpu.SemaphoreType.DMA,
          pltpu.SemaphoreType.DMA,
      ],
  )
  def sc_fn(z_hbm, lab_hbm, means_hbm, out_hbm,
            idx_v, z_all, rows_v0, rows_v1, rows_v2, rows_v3,
            rows_v4, rows_v5, rows_v6, part_v,
            gsem, zsem):
    wid = lax.axis_index("s") * NC + lax.axis_index("c")
    base = wid * BPW

    row_bufs = (rows_v0, rows_v1, rows_v2, rows_v3, rows_v4, rows_v5, rows_v6)
    NBUF = len(row_bufs)
    gcp = [None] * NCHUNK
    zcp = [None] * NCHUNK
    for c in range(NCHUNK):
      zcp[c] = pltpu.async_copy(
          z_hbm.at[pl.ds(base + c * CHUNK, CHUNK)],
          z_all.at[pl.ds(c * CHUNK, CHUNK)], zsem)
    pltpu.sync_copy(lab_hbm.at[pl.ds(base, BPW)], idx_v)
    for c in range(min(NBUF, NCHUNK)):
      gcp[c] = pltpu.async_copy(
          means_hbm.at[idx_v.at[pl.ds(c * CHUNK, CHUNK)]],
          row_bufs[c], gsem)

    zero = jnp.zeros((L,), jnp.float32)
    accs = (zero,) * VPR
    for c in range(NCHUNK):
      gcp[c].wait()
      zcp[c].wait()
      rows_v = row_bufs[c % NBUF]
      zoff = c * CHUNK

      def row_body(r, a, zoff=zoff, rows_v=rows_v):
        new = []
        for v in range(VPR):
          diff = z_all[zoff + r, pl.ds(v * L, L)] - rows_v[r, pl.ds(v * L, L)]
          new.append(a[v] + diff * diff)
        return tuple(new)

      accs = lax.fori_loop(0, CHUNK, row_body, accs)
      if c + NBUF < NCHUNK:
        gcp[c + NBUF] = pltpu.async_copy(
            means_hbm.at[idx_v.at[pl.ds((c + NBUF) * CHUNK, CHUNK)]],
            row_bufs[c % NBUF], gsem)
    sq = accs[0]
    for v in range(1, VPR):
      sq = sq + accs[v]

    part_v[...] = 0.5 * sq
    pltpu.sync_copy(part_v, out_hbm.at[wid])

  return sc_fn


_sc_fn = _make_sc_fn()


def kernel(z, sldj, labels, means):
  parts = _sc_fn(z, labels, means)
  const = 0.5 * D * math.log(2.0 * math.pi)
  return (parts.sum() - sldj.sum()) / B + const


# speedup vs baseline: 1.0116x; 1.0116x over previous
import functools
import math

import jax
import jax.numpy as jnp
from jax import lax
from jax.experimental import pallas as pl
from jax.experimental.pallas import tpu as pltpu
from jax.experimental.pallas import tpu_sc as plsc

B = 16384
D = 128
L = 16
NC = 2
NS = 16
NW = NC * NS
BPW = B // NW
CHUNK = 64
NCHUNK = BPW // CHUNK
VPR = D // L


def _make_sc_fn():
  mesh = plsc.VectorSubcoreMesh(core_axis_name="c", subcore_axis_name="s")

  @functools.partial(
      pl.kernel,
      mesh=mesh,
      out_type=jax.ShapeDtypeStruct((NW, L), jnp.float32),
      scratch_types=[
          pltpu.VMEM((BPW,), jnp.int32),
          pltpu.VMEM((BPW, D), jnp.float32),
          pltpu.VMEM((CHUNK, D), jnp.float32),
          pltpu.VMEM((CHUNK, D), jnp.float32),
          pltpu.VMEM((CHUNK, D), jnp.float32),
          pltpu.VMEM((CHUNK, D), jnp.float32),
          pltpu.VMEM((L,), jnp.float32),
          pltpu.SemaphoreType.DMA,
          pltpu.SemaphoreType.DMA,
      ],
  )
  def sc_fn(z_hbm, lab_hbm, means_hbm, out_hbm,
            idx_v, z_all, rows_v0, rows_v1, rows_v2, rows_v3, part_v,
            gsem, zsem):
    wid = lax.axis_index("s") * NC + lax.axis_index("c")
    base = wid * BPW

    row_bufs = (rows_v0, rows_v1, rows_v2, rows_v3)
    NBUF = len(row_bufs)
    gcp = [None] * NCHUNK
    zcp = [None] * NCHUNK
    for c in range(NCHUNK):
      zcp[c] = pltpu.async_copy(
          z_hbm.at[pl.ds(base + c * CHUNK, CHUNK)],
          z_all.at[pl.ds(c * CHUNK, CHUNK)], zsem)
    pltpu.sync_copy(lab_hbm.at[pl.ds(base, BPW)], idx_v)
    for c in range(min(NBUF, NCHUNK)):
      gcp[c] = pltpu.async_copy(
          means_hbm.at[idx_v.at[pl.ds(c * CHUNK, CHUNK)]],
          row_bufs[c], gsem)

    zero = jnp.zeros((L,), jnp.float32)
    accs = (zero,) * VPR
    for c in range(NCHUNK):
      gcp[c].wait()
      zcp[c].wait()
      rows_v = row_bufs[c % NBUF]
      zoff = c * CHUNK

      def row_body(r, a, zoff=zoff, rows_v=rows_v):
        new = []
        for v in range(VPR):
          diff = z_all[zoff + r, pl.ds(v * L, L)] - rows_v[r, pl.ds(v * L, L)]
          new.append(a[v] + diff * diff)
        return tuple(new)

      accs = lax.fori_loop(0, CHUNK, row_body, accs)
      if c + NBUF < NCHUNK:
        gcp[c + NBUF] = pltpu.async_copy(
            means_hbm.at[idx_v.at[pl.ds((c + NBUF) * CHUNK, CHUNK)]],
            row_bufs[c % NBUF], gsem)
    sq = accs[0]
    for v in range(1, VPR):
      sq = sq + accs[v]

    part_v[...] = 0.5 * sq
    pltpu.sync_copy(part_v, out_hbm.at[wid])

  return sc_fn


_sc_fn = _make_sc_fn()


def kernel(z, sldj, labels, means):
  parts = _sc_fn(z, labels, means)
  const = 0.5 * D * math.log(2.0 * math.pi)
  return (parts.sum() - sldj.sum()) / B + const
